# baseline (speedup 1.0000x reference)
import functools

import jax
import jax.numpy as jnp
from jax import lax
from jax.experimental import pallas as pl
from jax.experimental.pallas import tpu as pltpu
from jax.experimental.pallas import tpu_sc as plsc

N = 50000
E = 800000
NP = 50176
NC, NS = 2, 16
RPT = NP // NS
GSZ = 128
EP = 819200
NG = EP // GSZ
GB = 8
F32 = jnp.float32

@functools.cache
def _mesh():
    return plsc.VectorSubcoreMesh(
        core_axis_name="c", subcore_axis_name="s",
        num_cores=NC, num_subcores=NS)


def _sds(shape):
    return jax.ShapeDtypeStruct(shape, F32)


def _acc_init(zeros_hbm, accs, stage, s):
    r0 = s * RPT
    pltpu.sync_copy(zeros_hbm, stage)
    for acc in accs:
        pltpu.sync_copy(stage, acc.at[pl.ds(r0, RPT)])
    plsc.subcore_barrier()


def _acc_writeout(accs, outs, stage, s):
    plsc.subcore_barrier()
    r0 = s * RPT
    for acc, out in zip(accs, outs):
        pltpu.sync_copy(acc.at[pl.ds(r0, RPT)], stage)
        pltpu.sync_copy(stage, out.at[pl.ds(r0, RPT)])


def _edge_loop(srcg, dstg, src_v, dst_v, rows_v, base_grp, nblk,
               tab, acc, semg, sems, semi):
    pltpu.async_copy(srcg.at[pl.ds(base_grp, GB)], src_v.at[0], semi)
    pltpu.async_copy(dstg.at[pl.ds(base_grp, GB)], dst_v.at[0], semi)

    def blk(b, carry):
        bank = lax.rem(b, 2)
        g0 = base_grp + b * GB
        sv, dv = src_v.at[bank], dst_v.at[bank]
        pltpu.make_async_copy(srcg.at[pl.ds(g0, GB)], sv, semi).wait()
        pltpu.make_async_copy(dstg.at[pl.ds(g0, GB)], dv, semi).wait()

        @pl.when(b + 1 < nblk)
        def _():
            g1 = g0 + GB
            pltpu.async_copy(srcg.at[pl.ds(g1, GB)], src_v.at[1 - bank],
                             semi)
            pltpu.async_copy(dstg.at[pl.ds(g1, GB)], dst_v.at[1 - bank],
                             semi)

        dg = [pltpu.async_copy(tab.at[sv.at[j]], rows_v.at[j], semg.at[j])
              for j in range(GB)]
        ds = []
        for j in range(GB):
            dg[j].wait()
            ds.append(pltpu.async_copy(rows_v.at[j], acc.at[dv.at[j]],
                                       sems, add=True))
        for d in ds:
            d.wait()
        return carry
    lax.fori_loop(0, nblk, blk, 0)


def _deg_body(dstg, zeros1, ones, d0, d1, dst_v, ones_v, stage, acc,
              sems):
    c = lax.axis_index("c")
    s = lax.axis_index("s")
    _acc_init(zeros1, [acc], stage, s)
    pltpu.sync_copy(ones, ones_v)
    w = c * NS + s
    gpw = NG // (NC * NS)

    def blk(b, carry):
        g0 = w * gpw + b * GB
        pltpu.sync_copy(dstg.at[pl.ds(g0, GB)], dst_v)
        ds = [pltpu.async_copy(ones_v, acc.at[dst_v.at[j]], sems,
                               add=True)
              for j in range(GB)]
        for d in ds:
            d.wait()
        return carry
    lax.fori_loop(0, gpw // GB, blk, 0)

    @pl.when(c == 0)
    def _():
        _acc_writeout([acc], [d0], stage, s)

    @pl.when(c == 1)
    def _():
        _acc_writeout([acc], [d1], stage, s)


def _sc_degree(dstg, zeros1, ones):
    f = pl.kernel(
        _deg_body,
        out_type=[_sds((NP,)), _sds((NP,))],
        mesh=_mesh(),
        compiler_params=pltpu.CompilerParams(use_tc_tiling_on_sc=False),
        scratch_types=[
            pltpu.VMEM((GB, GSZ), jnp.int32),
            pltpu.VMEM((GSZ,), F32),
            pltpu.VMEM((RPT,), F32),
            pltpu.VMEM_SHARED((NP,), F32),
            pltpu.SemaphoreType.DMA,
        ],
    )
    return f(dstg, zeros1, ones)


def _stage_in(tab_hbm, acc, stage, s):
    r0 = s * RPT
    pltpu.sync_copy(tab_hbm.at[pl.ds(r0, RPT)], stage)
    pltpu.sync_copy(stage, acc.at[pl.ds(r0, RPT)])
    plsc.subcore_barrier()


def _split_body(srcg, dstg, tab, p0, p1,
                src_v, dst_v, rows_v, stage, acc, semg, sems, semi):
    c = lax.axis_index("c")
    s = lax.axis_index("s")
    _stage_in(tab, acc, stage, s)
    w = c * NS + s
    gpw = NG // (NC * NS)
    _edge_loop(srcg, dstg, src_v, dst_v, rows_v, w * gpw, gpw // GB,
               tab, acc, semg, sems, semi)

    @pl.when(c == 0)
    def _():
        _acc_writeout([acc], [p0], stage, s)

    @pl.when(c == 1)
    def _():
        _acc_writeout([acc], [p1], stage, s)


def _sc_propagate_split(srcg, dstg, tab):
    f = pl.kernel(
        _split_body,
        out_type=[_sds((NP, 16)), _sds((NP, 16))],
        mesh=_mesh(),
        compiler_params=pltpu.CompilerParams(use_tc_tiling_on_sc=False),
        scratch_types=[
            pltpu.VMEM((2, GB, GSZ), jnp.int32),
            pltpu.VMEM((2, GB, GSZ), jnp.int32),
            pltpu.VMEM((GB, GSZ, 16), F32),
            pltpu.VMEM((RPT, 16), F32),
            pltpu.VMEM_SHARED((NP, 16), F32),
            pltpu.SemaphoreType.DMA((GB,)),
            pltpu.SemaphoreType.DMA,
            pltpu.SemaphoreType.DMA,
        ],
    )
    return f(srcg, dstg, tab)


def _make_multi_body(k):
    def body(*refs):
        srcg, dstg = refs[0], refs[1]
        tabs = refs[2:2 + 2 * k]
        outs = refs[2 + 2 * k:2 + 4 * k]
        (src_v, dst_v, rows_v, stage, acc,
         semg, sems, semi) = refs[2 + 4 * k:]
        c = lax.axis_index("c")
        s = lax.axis_index("s")
        gpt = NG // NS

        def run(tab, out):
            _stage_in(tab, acc, stage, s)
            _edge_loop(srcg, dstg, src_v, dst_v, rows_v, s * gpt,
                       gpt // GB, tab, acc, semg, sems, semi)
            _acc_writeout([acc], [out], stage, s)

        @pl.when(c == 0)
        def _():
            for r in range(k):
                run(tabs[2 * r], outs[2 * r])

        @pl.when(c == 1)
        def _():
            for r in range(k):
                run(tabs[2 * r + 1], outs[2 * r + 1])

    return body


def _sc_propagate_multi(srcg, dstg, tabs):
    k = len(tabs) // 2
    f = pl.kernel(
        _make_multi_body(k),
        out_type=[_sds((NP, 16))] * (2 * k),
        mesh=_mesh(),
        compiler_params=pltpu.CompilerParams(use_tc_tiling_on_sc=False),
        scratch_types=[
            pltpu.VMEM((2, GB, GSZ), jnp.int32),
            pltpu.VMEM((2, GB, GSZ), jnp.int32),
            pltpu.VMEM((GB, GSZ, 16), F32),
            pltpu.VMEM((RPT, 16), F32),
            pltpu.VMEM_SHARED((NP, 16), F32),
            pltpu.SemaphoreType.DMA((GB,)),
            pltpu.SemaphoreType.DMA,
            pltpu.SemaphoreType.DMA,
        ],
    )
    return f(srcg, dstg, *tabs)


RB = 3136
TG = NP // RB


def _rows(w):
    return pl.BlockSpec((RB, w), lambda i: (i, 0))


def _full(a, b):
    return pl.BlockSpec((a, b), lambda i: (0, 0))


def _tc1_body(d0, d1, xp, dinv16_ref, g1_ref):
    deg = d0[...] + d1[...] + 1.0
    dinv16 = jnp.broadcast_to(lax.rsqrt(deg), (RB, 16))
    dinv16_ref[...] = dinv16
    g1_ref[...] = dinv16 * xp[...]


def _tc_prep(d0, d1, xp):
    return pl.pallas_call(
        _tc1_body,
        grid=(TG,),
        in_specs=[_rows(1), _rows(1), _rows(16)],
        out_specs=[_rows(16), _rows(16)],
        out_shape=[_sds((NP, 16)), _sds((NP, 16))],
    )(d0, d1, xp)


def _tc2_body(*refs):
    p0, p1, g1, dinv16_ref = refs[0:4]
    w1c, b1c = refs[4:8], refs[8:12]
    outs = refs[12:16]
    dinv16 = dinv16_ref[...]
    sx = dinv16 * (p0[...] + p1[...] - g1[...])
    for t in range(4):
        h = jnp.dot(sx, w1c[t][...], preferred_element_type=F32) + b1c[t][...]
        outs[t][...] = dinv16 * jnp.maximum(h, 0.0)


def _tc_layer1(p0, p1, g1, dinv16, w1c, b1c):
    return pl.pallas_call(
        _tc2_body,
        grid=(TG,),
        in_specs=[_rows(16)] * 4 + [_full(16, 16)] * 4 + [_full(1, 16)] * 4,
        out_specs=[_rows(16)] * 4,
        out_shape=[_sds((NP, 16))] * 4,
    )(p0, p1, g1, dinv16, *w1c, *b1c)


def _tc3_body(*refs):
    os_ = refs[0:4]
    dinv16_ref = refs[4]
    w2r, w3c = refs[5:9], refs[9:15]
    b2 = refs[15]
    outs = refs[16:22]
    dinv16 = dinv16_ref[...]
    h2 = b2[...]
    for t in range(4):
        z = dinv16 * os_[t][...]
        h2 = h2 + jnp.dot(z, w2r[t][...], preferred_element_type=F32)
    h2 = jnp.maximum(h2, 0.0)
    for u in range(6):
        m = jnp.dot(h2, w3c[u][...], preferred_element_type=F32)
        outs[u][...] = dinv16 * m


def _tc_layer2(os_, dinv16, w2r, w3c, b2):
    return pl.pallas_call(
        _tc3_body,
        grid=(TG,),
        in_specs=([_rows(16)] * 4 + [_rows(16)] + [_full(16, 128)] * 4
                  + [_full(128, 16)] * 6 + [_full(1, 128)]),
        out_specs=[_rows(16)] * 6,
        out_shape=[_sds((NP, 16))] * 6,
    )(*os_, dinv16, *w2r, *w3c, b2)


def _tc4_body(*refs):
    qs = refs[0:6]
    dinv16_ref = refs[6]
    b3c = refs[7:13]
    outs = refs[13:19]
    dinv16 = dinv16_ref[...]
    for t in range(6):
        outs[t][...] = dinv16 * qs[t][...] + b3c[t][...]


def _tc_final(qs, dinv16, b3c):
    return pl.pallas_call(
        _tc4_body,
        grid=(TG,),
        in_specs=[_rows(16)] * 6 + [_rows(16)] + [_full(1, 16)] * 6,
        out_specs=[_rows(16)] * 6,
        out_shape=[_sds((NP, 16))] * 6,
    )(*qs, dinv16, *b3c)


def kernel(x, edge_index, W1, b1, W2, b2, W3, b3):
    src = edge_index[0]
    dst = edge_index[1]
    npad = EP - E
    pad_src = jnp.full((npad,), N, jnp.int32)
    pad_dst = (N + (jnp.arange(npad, dtype=jnp.int32) % (NP - N)))
    srcg = jnp.concatenate([src, pad_src]).reshape(NG, GSZ)
    dstg = jnp.concatenate([dst, pad_dst]).reshape(NG, GSZ)

    xp = jnp.zeros((NP, 16), F32).at[:N, :12].set(x)
    w1p = jnp.zeros((16, 64), F32).at[:12].set(W1)
    w1c = [w1p[:, 16 * t:16 * (t + 1)] for t in range(4)]
    b1c = [b1.reshape(1, 64)[:, 16 * t:16 * (t + 1)] for t in range(4)]
    w2r = [W2[16 * t:16 * (t + 1), :] for t in range(4)]
    w3c = [W3[:, 16 * t:16 * (t + 1)] for t in range(6)]
    b3c = [b3.reshape(1, 96)[:, 16 * t:16 * (t + 1)] for t in range(6)]
    zeros1 = jnp.zeros((RPT,), F32)
    ones = jnp.ones((GSZ,), F32)

    d0, d1 = _sc_degree(dstg, zeros1, ones)
    dinv16, g1 = _tc_prep(d0.reshape(NP, 1), d1.reshape(NP, 1), xp)

    p0, p1 = _sc_propagate_split(srcg, dstg, g1)
    g2s = _tc_layer1(p0, p1, g1, dinv16, w1c, b1c)

    o2s = _sc_propagate_multi(srcg, dstg, list(g2s))
    g3s = _tc_layer2(list(o2s), dinv16, w2r, w3c, b2.reshape(1, 128))

    qs = _sc_propagate_multi(srcg, dstg, list(g3s))
    ys = _tc_final(list(qs), dinv16, b3c)
    y = jnp.concatenate(list(ys), axis=1)
    return y[:N].reshape(N, 8, 12)


# speedup vs baseline: 1.0261x; 1.0261x over previous
import functools

import jax
import jax.numpy as jnp
from jax import lax
from jax.experimental import pallas as pl
from jax.experimental.pallas import tpu as pltpu
from jax.experimental.pallas import tpu_sc as plsc

N = 50000
E = 800000
NP = 50176
NC, NS = 2, 16
RPT = NP // NS
GSZ = 128
EP = 819200
NG = EP // GSZ
GB = 8
F32 = jnp.float32

@functools.cache
def _mesh():
    return plsc.VectorSubcoreMesh(
        core_axis_name="c", subcore_axis_name="s",
        num_cores=NC, num_subcores=NS)


def _sds(shape):
    return jax.ShapeDtypeStruct(shape, F32)


def _acc_init(zeros_hbm, accs, stage, s):
    r0 = s * RPT
    pltpu.sync_copy(zeros_hbm, stage)
    for acc in accs:
        pltpu.sync_copy(stage, acc.at[pl.ds(r0, RPT)])
    plsc.subcore_barrier()


def _acc_writeout(accs, outs, stage, s):
    plsc.subcore_barrier()
    r0 = s * RPT
    for acc, out in zip(accs, outs):
        pltpu.sync_copy(acc.at[pl.ds(r0, RPT)], stage)
        pltpu.sync_copy(stage, out.at[pl.ds(r0, RPT)])


def _edge_loop(srcg, dstg, src_v, dst_v, rows_v, base_grp, nblk,
               tab, acc, semg, sems, semi):
    pltpu.async_copy(srcg.at[pl.ds(base_grp, GB)], src_v.at[0], semi)
    pltpu.async_copy(dstg.at[pl.ds(base_grp, GB)], dst_v.at[0], semi)

    def blk(b, carry):
        bank = lax.rem(b, 2)
        g0 = base_grp + b * GB
        sv, dv = src_v.at[bank], dst_v.at[bank]
        pltpu.make_async_copy(srcg.at[pl.ds(g0, GB)], sv, semi).wait()
        pltpu.make_async_copy(dstg.at[pl.ds(g0, GB)], dv, semi).wait()

        @pl.when(b + 1 < nblk)
        def _():
            g1 = g0 + GB
            pltpu.async_copy(srcg.at[pl.ds(g1, GB)], src_v.at[1 - bank],
                             semi)
            pltpu.async_copy(dstg.at[pl.ds(g1, GB)], dst_v.at[1 - bank],
                             semi)

        dg = [pltpu.async_copy(tab.at[sv.at[j]], rows_v.at[j], semg.at[j])
              for j in range(GB)]
        ds = []
        for j in range(GB):
            dg[j].wait()
            ds.append(pltpu.async_copy(rows_v.at[j], acc.at[dv.at[j]],
                                       sems, add=True))
        for d in ds:
            d.wait()
        return carry
    lax.fori_loop(0, nblk, blk, 0)


def _deg_body(dstg, zeros1, ones, d0, d1, dst_v, ones_v, stage, acc,
              sems):
    c = lax.axis_index("c")
    s = lax.axis_index("s")
    _acc_init(zeros1, [acc], stage, s)
    pltpu.sync_copy(ones, ones_v)
    w = c * NS + s
    gpw = NG // (NC * NS)

    def blk(b, carry):
        g0 = w * gpw + b * GB
        pltpu.sync_copy(dstg.at[pl.ds(g0, GB)], dst_v)
        ds = [pltpu.async_copy(ones_v, acc.at[dst_v.at[j]], sems,
                               add=True)
              for j in range(GB)]
        for d in ds:
            d.wait()
        return carry
    lax.fori_loop(0, gpw // GB, blk, 0)

    @pl.when(c == 0)
    def _():
        _acc_writeout([acc], [d0], stage, s)

    @pl.when(c == 1)
    def _():
        _acc_writeout([acc], [d1], stage, s)


def _sc_degree(dstg, zeros1, ones):
    f = pl.kernel(
        _deg_body,
        out_type=[_sds((NP,)), _sds((NP,))],
        mesh=_mesh(),
        compiler_params=pltpu.CompilerParams(use_tc_tiling_on_sc=False),
        scratch_types=[
            pltpu.VMEM((GB, GSZ), jnp.int32),
            pltpu.VMEM((GSZ,), F32),
            pltpu.VMEM((RPT,), F32),
            pltpu.VMEM_SHARED((NP,), F32),
            pltpu.SemaphoreType.DMA,
        ],
    )
    return f(dstg, zeros1, ones)


def _stage_in(tab_hbm, acc, stage, s):
    r0 = s * RPT
    pltpu.sync_copy(tab_hbm.at[pl.ds(r0, RPT)], stage)
    pltpu.sync_copy(stage, acc.at[pl.ds(r0, RPT)])
    plsc.subcore_barrier()


def _split_body(srcg, dstg, tab, p0, p1,
                src_v, dst_v, rows_v, stage, acc, semg, sems, semi):
    c = lax.axis_index("c")
    s = lax.axis_index("s")
    _stage_in(tab, acc, stage, s)
    w = c * NS + s
    gpw = NG // (NC * NS)
    _edge_loop(srcg, dstg, src_v, dst_v, rows_v, w * gpw, gpw // GB,
               tab, acc, semg, sems, semi)

    @pl.when(c == 0)
    def _():
        _acc_writeout([acc], [p0], stage, s)

    @pl.when(c == 1)
    def _():
        _acc_writeout([acc], [p1], stage, s)


def _sc_propagate_split(srcg, dstg, tab):
    f = pl.kernel(
        _split_body,
        out_type=[_sds((NP, 16)), _sds((NP, 16))],
        mesh=_mesh(),
        compiler_params=pltpu.CompilerParams(use_tc_tiling_on_sc=False),
        scratch_types=[
            pltpu.VMEM((2, GB, GSZ), jnp.int32),
            pltpu.VMEM((2, GB, GSZ), jnp.int32),
            pltpu.VMEM((GB, GSZ, 16), F32),
            pltpu.VMEM((RPT, 16), F32),
            pltpu.VMEM_SHARED((NP, 16), F32),
            pltpu.SemaphoreType.DMA((GB,)),
            pltpu.SemaphoreType.DMA,
            pltpu.SemaphoreType.DMA,
        ],
    )
    return f(srcg, dstg, tab)


def _pair_body(srcg, dstg, t0, t1, o0, o1,
               src_v, dst_v, rows_v, stage, acc, semg, sems, semi):
    c = lax.axis_index("c")
    s = lax.axis_index("s")
    gpt = NG // NS

    def run(tab, out):
        _stage_in(tab, acc, stage, s)
        _edge_loop(srcg, dstg, src_v, dst_v, rows_v, s * gpt, gpt // GB,
                   tab, acc, semg, sems, semi)
        _acc_writeout([acc], [out], stage, s)

    @pl.when(c == 0)
    def _():
        run(t0, o0)

    @pl.when(c == 1)
    def _():
        run(t1, o1)


def _sc_propagate_pair(srcg, dstg, t0, t1):
    f = pl.kernel(
        _pair_body,
        out_type=[_sds((NP, 16)), _sds((NP, 16))],
        mesh=_mesh(),
        compiler_params=pltpu.CompilerParams(use_tc_tiling_on_sc=False),
        scratch_types=[
            pltpu.VMEM((2, GB, GSZ), jnp.int32),
            pltpu.VMEM((2, GB, GSZ), jnp.int32),
            pltpu.VMEM((GB, GSZ, 16), F32),
            pltpu.VMEM((RPT, 16), F32),
            pltpu.VMEM_SHARED((NP, 16), F32),
            pltpu.SemaphoreType.DMA((GB,)),
            pltpu.SemaphoreType.DMA,
            pltpu.SemaphoreType.DMA,
        ],
    )
    return f(srcg, dstg, t0, t1)


RB = 3136
TG = NP // RB


def _rows(w):
    return pl.BlockSpec((RB, w), lambda i: (i, 0))


def _full(a, b):
    return pl.BlockSpec((a, b), lambda i: (0, 0))


def _tc1_body(d0, d1, xp, dinv16_ref, g1_ref):
    deg = d0[...] + d1[...] + 1.0
    dinv16 = jnp.broadcast_to(lax.rsqrt(deg), (RB, 16))
    dinv16_ref[...] = dinv16
    g1_ref[...] = dinv16 * xp[...]


def _tc_prep(d0, d1, xp):
    return pl.pallas_call(
        _tc1_body,
        grid=(TG,),
        in_specs=[_rows(1), _rows(1), _rows(16)],
        out_specs=[_rows(16), _rows(16)],
        out_shape=[_sds((NP, 16)), _sds((NP, 16))],
    )(d0, d1, xp)


def _tc2_body(*refs):
    p0, p1, g1, dinv16_ref = refs[0:4]
    w1c, b1c = refs[4:8], refs[8:12]
    outs = refs[12:16]
    dinv16 = dinv16_ref[...]
    sx = dinv16 * (p0[...] + p1[...] - g1[...])
    for t in range(4):
        h = jnp.dot(sx, w1c[t][...], preferred_element_type=F32) + b1c[t][...]
        outs[t][...] = dinv16 * jnp.maximum(h, 0.0)


def _tc_layer1(p0, p1, g1, dinv16, w1c, b1c):
    return pl.pallas_call(
        _tc2_body,
        grid=(TG,),
        in_specs=[_rows(16)] * 4 + [_full(16, 16)] * 4 + [_full(1, 16)] * 4,
        out_specs=[_rows(16)] * 4,
        out_shape=[_sds((NP, 16))] * 4,
    )(p0, p1, g1, dinv16, *w1c, *b1c)


def _tc3_body(*refs):
    os_ = refs[0:4]
    dinv16_ref = refs[4]
    w2r, w3c = refs[5:9], refs[9:15]
    b2 = refs[15]
    outs = refs[16:22]
    dinv16 = dinv16_ref[...]
    h2 = b2[...]
    for t in range(4):
        z = dinv16 * os_[t][...]
        h2 = h2 + jnp.dot(z, w2r[t][...], preferred_element_type=F32)
    h2 = jnp.maximum(h2, 0.0)
    for u in range(6):
        m = jnp.dot(h2, w3c[u][...], preferred_element_type=F32)
        outs[u][...] = dinv16 * m


def _tc_layer2(os_, dinv16, w2r, w3c, b2):
    return pl.pallas_call(
        _tc3_body,
        grid=(TG,),
        in_specs=([_rows(16)] * 4 + [_rows(16)] + [_full(16, 128)] * 4
                  + [_full(128, 16)] * 6 + [_full(1, 128)]),
        out_specs=[_rows(16)] * 6,
        out_shape=[_sds((NP, 16))] * 6,
    )(*os_, dinv16, *w2r, *w3c, b2)


def _tc4_body(*refs):
    qs = refs[0:6]
    dinv16_ref = refs[6]
    b3c = refs[7:13]
    outs = refs[13:19]
    dinv16 = dinv16_ref[...]
    for t in range(6):
        outs[t][...] = dinv16 * qs[t][...] + b3c[t][...]


def _tc_final(qs, dinv16, b3c):
    return pl.pallas_call(
        _tc4_body,
        grid=(TG,),
        in_specs=[_rows(16)] * 6 + [_rows(16)] + [_full(1, 16)] * 6,
        out_specs=[_rows(16)] * 6,
        out_shape=[_sds((NP, 16))] * 6,
    )(*qs, dinv16, *b3c)


def kernel(x, edge_index, W1, b1, W2, b2, W3, b3):
    src = edge_index[0]
    dst = edge_index[1]
    npad = EP - E
    pad_src = jnp.full((npad,), N, jnp.int32)
    pad_dst = (N + (jnp.arange(npad, dtype=jnp.int32) % (NP - N)))
    srcg = jnp.concatenate([src, pad_src]).reshape(NG, GSZ)
    dstg = jnp.concatenate([dst, pad_dst]).reshape(NG, GSZ)

    xp = jnp.zeros((NP, 16), F32).at[:N, :12].set(x)
    w1p = jnp.zeros((16, 64), F32).at[:12].set(W1)
    w1c = [w1p[:, 16 * t:16 * (t + 1)] for t in range(4)]
    b1c = [b1.reshape(1, 64)[:, 16 * t:16 * (t + 1)] for t in range(4)]
    w2r = [W2[16 * t:16 * (t + 1), :] for t in range(4)]
    w3c = [W3[:, 16 * t:16 * (t + 1)] for t in range(6)]
    b3c = [b3.reshape(1, 96)[:, 16 * t:16 * (t + 1)] for t in range(6)]
    zeros1 = jnp.zeros((RPT,), F32)
    ones = jnp.ones((GSZ,), F32)

    d0, d1 = _sc_degree(dstg, zeros1, ones)
    dinv16, g1 = _tc_prep(d0.reshape(NP, 1), d1.reshape(NP, 1), xp)

    p0, p1 = _sc_propagate_split(srcg, dstg, g1)
    g2s = _tc_layer1(p0, p1, g1, dinv16, w1c, b1c)

    o2a = _sc_propagate_pair(srcg, dstg, g2s[0], g2s[1])
    o2b = _sc_propagate_pair(srcg, dstg, g2s[2], g2s[3])
    g3s = _tc_layer2(list(o2a) + list(o2b), dinv16, w2r, w3c,
                     b2.reshape(1, 128))

    qa = _sc_propagate_pair(srcg, dstg, g3s[0], g3s[1])
    qb = _sc_propagate_pair(srcg, dstg, g3s[2], g3s[3])
    qc = _sc_propagate_pair(srcg, dstg, g3s[4], g3s[5])
    ys = _tc_final(list(qa) + list(qb) + list(qc), dinv16, b3c)
    y = jnp.concatenate(list(ys), axis=1)
    return y[:N].reshape(N, 8, 12)


# speedup vs baseline: 1.0930x; 1.0653x over previous
import functools

import jax
import jax.numpy as jnp
from jax import lax
from jax.experimental import pallas as pl
from jax.experimental.pallas import tpu as pltpu
from jax.experimental.pallas import tpu_sc as plsc

N = 50000
E = 800000
NP = 50176
NC, NS = 2, 16
RPT = NP // NS
GSZ = 128
EP = 819200
NG = EP // GSZ
GB = 8
F32 = jnp.float32

@functools.cache
def _mesh():
    return plsc.VectorSubcoreMesh(
        core_axis_name="c", subcore_axis_name="s",
        num_cores=NC, num_subcores=NS)


def _sds(shape):
    return jax.ShapeDtypeStruct(shape, F32)


def _acc_init(zeros_hbm, accs, stage, s):
    r0 = s * RPT
    pltpu.sync_copy(zeros_hbm, stage)
    for acc in accs:
        pltpu.sync_copy(stage, acc.at[pl.ds(r0, RPT)])
    plsc.subcore_barrier()


def _acc_writeout(accs, outs, stage, s):
    plsc.subcore_barrier()
    r0 = s * RPT
    for acc, out in zip(accs, outs):
        pltpu.sync_copy(acc.at[pl.ds(r0, RPT)], stage)
        pltpu.sync_copy(stage, out.at[pl.ds(r0, RPT)])


def _edge_loop(srcg, dstg, src_v, dst_v, rows_v, base_grp, nblk,
               tab, acc, semg, sems, semi):
    pltpu.async_copy(srcg.at[pl.ds(base_grp, GB)], src_v.at[0], semi)
    pltpu.async_copy(dstg.at[pl.ds(base_grp, GB)], dst_v.at[0], semi)

    def blk(b, carry):
        bank = lax.rem(b, 2)
        g0 = base_grp + b * GB
        sv, dv = src_v.at[bank], dst_v.at[bank]
        pltpu.make_async_copy(srcg.at[pl.ds(g0, GB)], sv, semi).wait()
        pltpu.make_async_copy(dstg.at[pl.ds(g0, GB)], dv, semi).wait()

        @pl.when(b + 1 < nblk)
        def _():
            g1 = g0 + GB
            pltpu.async_copy(srcg.at[pl.ds(g1, GB)], src_v.at[1 - bank],
                             semi)
            pltpu.async_copy(dstg.at[pl.ds(g1, GB)], dst_v.at[1 - bank],
                             semi)

        dg = [pltpu.async_copy(tab.at[sv.at[j]], rows_v.at[j], semg.at[j])
              for j in range(GB)]
        ds = []
        for j in range(GB):
            dg[j].wait()
            ds.append(pltpu.async_copy(rows_v.at[j], acc.at[dv.at[j]],
                                       sems, add=True))
        for d in ds:
            d.wait()
        return carry
    lax.fori_loop(0, nblk, blk, 0)


def _deg_body(dstg, zeros1, ones, d0, d1, dst_v, ones_v, stage, acc,
              sems):
    c = lax.axis_index("c")
    s = lax.axis_index("s")
    _acc_init(zeros1, [acc], stage, s)
    pltpu.sync_copy(ones, ones_v)
    w = c * NS + s
    gpw = NG // (NC * NS)

    def blk(b, carry):
        g0 = w * gpw + b * GB
        pltpu.sync_copy(dstg.at[pl.ds(g0, GB)], dst_v)
        ds = [pltpu.async_copy(ones_v, acc.at[dst_v.at[j]], sems,
                               add=True)
              for j in range(GB)]
        for d in ds:
            d.wait()
        return carry
    lax.fori_loop(0, gpw // GB, blk, 0)

    @pl.when(c == 0)
    def _():
        _acc_writeout([acc], [d0], stage, s)

    @pl.when(c == 1)
    def _():
        _acc_writeout([acc], [d1], stage, s)


def _sc_degree(dstg, zeros1, ones):
    f = pl.kernel(
        _deg_body,
        out_type=[_sds((NP,)), _sds((NP,))],
        mesh=_mesh(),
        compiler_params=pltpu.CompilerParams(use_tc_tiling_on_sc=False),
        scratch_types=[
            pltpu.VMEM((GB, GSZ), jnp.int32),
            pltpu.VMEM((GSZ,), F32),
            pltpu.VMEM((RPT,), F32),
            pltpu.VMEM_SHARED((NP,), F32),
            pltpu.SemaphoreType.DMA,
        ],
    )
    return f(dstg, zeros1, ones)


def _stage_in(tab_hbm, acc, stage, s):
    r0 = s * RPT
    pltpu.sync_copy(tab_hbm.at[pl.ds(r0, RPT)], stage)
    pltpu.sync_copy(stage, acc.at[pl.ds(r0, RPT)])
    plsc.subcore_barrier()


def _split_body(srcg, dstg, tab, p0, p1,
                src_v, dst_v, rows_v, stage, acc, semg, sems, semi):
    c = lax.axis_index("c")
    s = lax.axis_index("s")
    _stage_in(tab, acc, stage, s)
    w = c * NS + s
    gpw = NG // (NC * NS)
    _edge_loop(srcg, dstg, src_v, dst_v, rows_v, w * gpw, gpw // GB,
               tab, acc, semg, sems, semi)

    @pl.when(c == 0)
    def _():
        _acc_writeout([acc], [p0], stage, s)

    @pl.when(c == 1)
    def _():
        _acc_writeout([acc], [p1], stage, s)


def _sc_propagate_split(srcg, dstg, tab):
    f = pl.kernel(
        _split_body,
        out_type=[_sds((NP, 16)), _sds((NP, 16))],
        mesh=_mesh(),
        compiler_params=pltpu.CompilerParams(use_tc_tiling_on_sc=False),
        scratch_types=[
            pltpu.VMEM((2, GB, GSZ), jnp.int32),
            pltpu.VMEM((2, GB, GSZ), jnp.int32),
            pltpu.VMEM((GB, GSZ, 16), F32),
            pltpu.VMEM((RPT, 16), F32),
            pltpu.VMEM_SHARED((NP, 16), F32),
            pltpu.SemaphoreType.DMA((GB,)),
            pltpu.SemaphoreType.DMA,
            pltpu.SemaphoreType.DMA,
        ],
    )
    return f(srcg, dstg, tab)


def _pair_body(srcg, dstg, t0, t1, o0, o1,
               src_v, dst_v, rows_v, stage, acc, semg, sems, semi):
    c = lax.axis_index("c")
    s = lax.axis_index("s")
    gpt = NG // NS

    def run(tab, out):
        _stage_in(tab, acc, stage, s)
        _edge_loop(srcg, dstg, src_v, dst_v, rows_v, s * gpt, gpt // GB,
                   tab, acc, semg, sems, semi)
        _acc_writeout([acc], [out], stage, s)

    @pl.when(c == 0)
    def _():
        run(t0, o0)

    @pl.when(c == 1)
    def _():
        run(t1, o1)


def _sc_propagate_pair(srcg, dstg, t0, t1):
    f = pl.kernel(
        _pair_body,
        out_type=[_sds((NP, 16)), _sds((NP, 16))],
        mesh=_mesh(),
        compiler_params=pltpu.CompilerParams(use_tc_tiling_on_sc=False),
        scratch_types=[
            pltpu.VMEM((2, GB, GSZ), jnp.int32),
            pltpu.VMEM((2, GB, GSZ), jnp.int32),
            pltpu.VMEM((GB, GSZ, 16), F32),
            pltpu.VMEM((RPT, 16), F32),
            pltpu.VMEM_SHARED((NP, 16), F32),
            pltpu.SemaphoreType.DMA((GB,)),
            pltpu.SemaphoreType.DMA,
            pltpu.SemaphoreType.DMA,
        ],
    )
    return f(srcg, dstg, t0, t1)


RB = 3136
TG = NP // RB


def _rows(w):
    return pl.BlockSpec((RB, w), lambda i: (i, 0))


def _full(a, b):
    return pl.BlockSpec((a, b), lambda i: (0, 0))


def _tc1_body(d0, d1, xp, dinv_ref, g1_ref):
    deg = d0[...] + d1[...] + 1.0
    dinv = lax.rsqrt(deg)
    dinv_ref[...] = dinv
    g1_ref[...] = dinv * xp[...]


def _tc_prep(d0, d1, xp):
    return pl.pallas_call(
        _tc1_body,
        grid=(TG,),
        in_specs=[_rows(1), _rows(1), _rows(16)],
        out_specs=[_rows(1), _rows(16)],
        out_shape=[_sds((NP, 1)), _sds((NP, 16))],
    )(d0, d1, xp)


def _tc2_body(p0, p1, g1, dinv_ref, w1, b1, *outs):
    dinv = dinv_ref[...]
    sx = dinv * (p0[...] + p1[...] - g1[...])
    h = jnp.dot(sx, w1[...], preferred_element_type=F32) + b1[...]
    g2 = dinv * jnp.maximum(h, 0.0)
    for t in range(4):
        outs[t][...] = g2[:, 16 * t:16 * (t + 1)]


def _tc_layer1(p0, p1, g1, dinv, w1p, b1):
    return pl.pallas_call(
        _tc2_body,
        grid=(TG,),
        in_specs=[_rows(16), _rows(16), _rows(16), _rows(1),
                  _full(16, 64), _full(1, 64)],
        out_specs=[_rows(16)] * 4,
        out_shape=[_sds((NP, 16))] * 4,
    )(p0, p1, g1, dinv, w1p, b1)


def _tc3_body(*refs):
    os_ = refs[0:4]
    dinv_ref, w2, w3, b2 = refs[4:8]
    outs = refs[8:14]
    dinv = dinv_ref[...]
    sp = jnp.concatenate([os_[t][...] for t in range(4)], axis=1)
    h2 = jnp.maximum(
        jnp.dot(dinv * sp, w2[...], preferred_element_type=F32) + b2[...],
        0.0)
    m = jnp.dot(h2, w3[...], preferred_element_type=F32)
    g3 = dinv * m
    for t in range(6):
        outs[t][...] = g3[:, 16 * t:16 * (t + 1)]


def _tc_layer2(os_, dinv, w2, w3, b2):
    return pl.pallas_call(
        _tc3_body,
        grid=(TG,),
        in_specs=[_rows(16)] * 4 + [_rows(1), _full(64, 128),
                                    _full(128, 96), _full(1, 128)],
        out_specs=[_rows(16)] * 6,
        out_shape=[_sds((NP, 16))] * 6,
    )(*os_, dinv, w2, w3, b2)


def _tc4_body(*refs):
    qs = refs[0:6]
    dinv_ref, b3 = refs[6], refs[7]
    out = refs[8]
    y = jnp.concatenate([qs[t][...] for t in range(6)], axis=1)
    out[...] = dinv_ref[...] * y + b3[...]


def _tc_final(qs, dinv, b3):
    return pl.pallas_call(
        _tc4_body,
        grid=(TG,),
        in_specs=[_rows(16)] * 6 + [_rows(1), _full(1, 96)],
        out_specs=_rows(96),
        out_shape=_sds((NP, 96)),
    )(*qs, dinv, b3)


def kernel(x, edge_index, W1, b1, W2, b2, W3, b3):
    src = edge_index[0]
    dst = edge_index[1]
    npad = EP - E
    pad_src = jnp.full((npad,), N, jnp.int32)
    pad_dst = (N + (jnp.arange(npad, dtype=jnp.int32) % (NP - N)))
    srcg = jnp.concatenate([src, pad_src]).reshape(NG, GSZ)
    dstg = jnp.concatenate([dst, pad_dst]).reshape(NG, GSZ)

    xp = jnp.zeros((NP, 16), F32).at[:N, :12].set(x)
    w1p = jnp.zeros((16, 64), F32).at[:12].set(W1)
    zeros1 = jnp.zeros((RPT,), F32)
    ones = jnp.ones((GSZ,), F32)

    d0, d1 = _sc_degree(dstg, zeros1, ones)
    dinv, g1 = _tc_prep(d0.reshape(NP, 1), d1.reshape(NP, 1), xp)

    p0, p1 = _sc_propagate_split(srcg, dstg, g1)
    g2s = _tc_layer1(p0, p1, g1, dinv, w1p, b1.reshape(1, 64))

    o2a = _sc_propagate_pair(srcg, dstg, g2s[0], g2s[1])
    o2b = _sc_propagate_pair(srcg, dstg, g2s[2], g2s[3])
    g3s = _tc_layer2(list(o2a) + list(o2b), dinv, W2, W3,
                     b2.reshape(1, 128))

    qa = _sc_propagate_pair(srcg, dstg, g3s[0], g3s[1])
    qb = _sc_propagate_pair(srcg, dstg, g3s[2], g3s[3])
    qc = _sc_propagate_pair(srcg, dstg, g3s[4], g3s[5])

    y = _tc_final(list(qa) + list(qb) + list(qc), dinv,
                  b3.reshape(1, 96))
    return y[:N].reshape(N, 8, 12)
